# baseline (speedup 1.0000x reference)
import functools

import jax
import jax.numpy as jnp
from jax import lax
from jax.experimental import pallas as pl
from jax.experimental.pallas import tpu as pltpu
from jax.experimental.pallas import tpu_sc as plsc

NX, NY = 432, 496
S = NX * NY
NV, MP, CP = 16000, 32, 4
CO = 64
EPS = 1e-3
VX, VY, VZ = 0.16, 0.16, 4.0
XOFF = VX / 2 + 0.0
YOFF = VY / 2 + (-39.68)
ZOFF = VZ / 2 + (-3.0)

BLK = 200
NBLK = NV // BLK

BANDS = 62
SS = S // BANDS
NTILE = 32
SCCH = 800
PCH = 256
LISTCAP = 16128
HIGH = lax.Precision.HIGHEST


def _features(vox, coords, nump):
    if nump.ndim == 2:
        nump = nump[:, 0]
    numf = nump.astype(jnp.float32)
    xyz = vox[:, :, :3]
    pmean = jnp.sum(xyz, axis=1, keepdims=True) / numf[:, None, None]
    f_cluster = xyz - pmean
    cf = coords.astype(jnp.float32)
    cen = jnp.stack([cf[:, 3] * VX + XOFF,
                     cf[:, 2] * VY + YOFF,
                     cf[:, 1] * VZ + ZOFF], axis=-1)
    f_center = xyz - cen[:, None, :]
    f = jnp.concatenate([vox, f_cluster, f_center], axis=-1)
    msk = nump[:, None] > lax.broadcasted_iota(jnp.int32, (1, MP), 1)
    return f * msk.astype(jnp.float32)[:, :, None]


def _stats_kernel(vox_ref, coords_ref, nump_ref, g_ref):
    f = _features(vox_ref[...], coords_ref[...], nump_ref[...])
    ones = jnp.ones(f.shape[:2] + (1,), jnp.float32)
    f1 = jnp.concatenate([f, ones], axis=-1).reshape(BLK * MP, 11)
    g = lax.dot_general(f1, f1, (((0,), (0,)), ((), ())),
                        precision=HIGH, preferred_element_type=jnp.float32)

    @pl.when(pl.program_id(0) == 0)
    def _():
        g_ref[...] = jnp.zeros_like(g_ref)

    g_ref[...] += g


def _apply_kernel(vox_ref, coords_ref, nump_ref, g_ref, w_ref, gam_ref,
                  bet_ref, pf_ref, key_ref):
    G = g_ref[...]
    W = w_ref[...]
    cnt = G[10, 10]
    svec = G[10, :10]
    mu = jnp.sum(W * svec[None, :], axis=1) / cnt
    WG = lax.dot_general(W, G[:10, :10], (((1,), (0,)), ((), ())),
                         precision=HIGH, preferred_element_type=jnp.float32)
    e2 = jnp.sum(WG * W, axis=1) / cnt
    var = e2 - mu * mu
    scale = gam_ref[...] * lax.rsqrt(var + EPS)
    shift = bet_ref[...] - mu * scale

    f = _features(vox_ref[...], coords_ref[...], nump_ref[...])
    h = lax.dot_general(f.reshape(BLK * MP, 10), W, (((1,), (1,)), ((), ())),
                        precision=HIGH, preferred_element_type=jnp.float32)
    h = h.reshape(BLK, MP, CO)
    h = jnp.maximum(h * scale[None, None, :] + shift[None, None, :], 0.0)
    pf_ref[...] = jnp.concatenate(
        [jnp.max(h, axis=1), jnp.zeros((BLK, 128 - CO), jnp.float32)], axis=1)
    c = coords_ref[...]
    key_ref[...] = (c[:, 0] * S + c[:, 1] + c[:, 2] * NX + c[:, 3])[:, None]


def _pfn(voxels, coords, voxel_num_points, W, gamma, beta):
    G = pl.pallas_call(
        _stats_kernel,
        grid=(NBLK,),
        in_specs=[pl.BlockSpec((BLK, MP, CP), lambda i: (i, 0, 0)),
                  pl.BlockSpec((BLK, 4), lambda i: (i, 0)),
                  pl.BlockSpec((BLK, 1), lambda i: (i, 0))],
        out_specs=pl.BlockSpec((11, 11), lambda i: (0, 0)),
        out_shape=jax.ShapeDtypeStruct((11, 11), jnp.float32),
    )(voxels, coords, voxel_num_points.reshape(NV, 1))

    pf, key = pl.pallas_call(
        _apply_kernel,
        grid=(NBLK,),
        in_specs=[pl.BlockSpec((BLK, MP, CP), lambda i: (i, 0, 0)),
                  pl.BlockSpec((BLK, 4), lambda i: (i, 0)),
                  pl.BlockSpec((BLK, 1), lambda i: (i, 0)),
                  pl.BlockSpec((11, 11), lambda i: (0, 0)),
                  pl.BlockSpec((CO, 10), lambda i: (0, 0)),
                  pl.BlockSpec((CO,), lambda i: (0,)),
                  pl.BlockSpec((CO,), lambda i: (0,))],
        out_specs=[pl.BlockSpec((BLK, 128), lambda i: (i, 0)),
                   pl.BlockSpec((BLK, 1), lambda i: (i, 0))],
        out_shape=[jax.ShapeDtypeStruct((NV, 128), jnp.float32),
                   jax.ShapeDtypeStruct((NV, 1), jnp.int32)],
    )(voxels, coords, voxel_num_points.reshape(NV, 1), G, W, gamma, beta)
    return pf, key.reshape(NV)


def _sc_scatter_kernel(pf_hbm, key_hbm, out_hbm, key_v, ids_v, offs_v,
                       idx_v, stage_v, buf_v, sem):
    wid = lax.axis_index("s") * 2 + lax.axis_index("c")
    iota = lax.iota(jnp.int32, 16)
    z16 = jnp.zeros((16,), jnp.float32)

    for c in range(16):
        def zrow(j, _, c=c):
            buf_v[c, pl.ds(j * 16, 16)] = z16
            return 0
        lax.fori_loop(0, SS // 16, zrow, 0)

    def bin_body(t, _):
        bin_id = wid + t * NTILE

        @pl.when(bin_id < 2 * BANDS)
        def _():
            _do_bin(bin_id, pf_hbm, key_hbm, out_hbm, key_v, ids_v, offs_v,
                    idx_v, stage_v, buf_v, sem)
        return 0

    lax.fori_loop(0, pl.cdiv(2 * BANDS, NTILE), bin_body, 0)


def _do_bin(bin_id, pf_hbm, key_hbm, out_hbm, key_v, ids_v, offs_v,
            idx_v, stage_v, buf_v, sem):
        iota = lax.iota(jnp.int32, 16)
        z16 = jnp.zeros((16,), jnp.float32)
        b = bin_id // BANDS
        band = bin_id % BANDS
        base = b * S + band * SS

        def scan_chunk(kc, cnt):
            pltpu.sync_copy(key_hbm.at[pl.ds(kc * SCCH, SCCH)], key_v)

            def scan_vec(i, cnt):
                kv = key_v[pl.ds(i * 16, 16)]
                rel = kv - base
                m = (rel >= 0) & (rel < SS)
                idvec = kc * SCCH + i * 16 + iota
                mi = m.astype(jnp.int32)
                pos = cnt - 1 + plsc.cumsum(mi)
                plsc.store_scatter(ids_v, [pos], idvec, mask=m)
                plsc.store_scatter(offs_v, [pos], rel, mask=m)
                return cnt + jnp.sum(mi)

            return lax.fori_loop(0, SCCH // 16, scan_vec, cnt)

        n = lax.fori_loop(0, NV // SCCH, scan_chunk, 0)
        nch = (n + PCH - 1) // PCH

        def cg_body(cg, _):
            def chunk_body(ch, _):
                c0 = ch * PCH
                m = jnp.minimum(PCH, n - c0)

                def bidx(j, _):
                    v = ids_v[pl.ds(c0 + j * 16, 16)]
                    v = jnp.minimum(jnp.maximum(v, 0), NV - 1)
                    idx_v[pl.ds(j * 16, 16)] = v
                    return 0

                lax.fori_loop(0, PCH // 16, bidx, 0)
                pltpu.async_copy(pf_hbm.at[idx_v], stage_v, sem).wait()

                def pillar(p, _):
                    osp = plsc.load_gather(
                        offs_v, [jnp.full((16,), c0 + p, jnp.int32)])
                    row = plsc.load_gather(
                        stage_v, [jnp.full((16,), p, jnp.int32), cg * 16 + iota])
                    plsc.store_scatter(buf_v, [iota, osp], row)
                    return 0

                lax.fori_loop(0, m, pillar, 0)
                return 0

            lax.fori_loop(0, nch, chunk_body, 0)

            row0 = b * CO + cg * 16
            pltpu.sync_copy(buf_v,
                            out_hbm.at[pl.ds(row0, 16), pl.ds(band * SS, SS)])

            def rchunk(ch, _):
                c0 = ch * PCH
                m = jnp.minimum(PCH, n - c0)

                def rpillar(p, _):
                    osp = plsc.load_gather(
                        offs_v, [jnp.full((16,), c0 + p, jnp.int32)])
                    plsc.store_scatter(buf_v, [iota, osp], z16)
                    return 0

                lax.fori_loop(0, m, rpillar, 0)
                return 0

            lax.fori_loop(0, nch, rchunk, 0)
            return 0

        lax.fori_loop(0, 4, cg_body, 0)


def _sc_scatter(pf16, key):
    mesh = plsc.VectorSubcoreMesh(core_axis_name="c", subcore_axis_name="s")
    run = functools.partial(
        pl.kernel, mesh=mesh,
        compiler_params=pltpu.CompilerParams(needs_layout_passes=False),
        out_type=jax.ShapeDtypeStruct((2 * CO, S), jnp.float32),
        scratch_types=[pltpu.VMEM((SCCH,), jnp.int32),
                       pltpu.VMEM((LISTCAP,), jnp.int32),
                       pltpu.VMEM((LISTCAP,), jnp.int32),
                       pltpu.VMEM((PCH,), jnp.int32),
                       pltpu.VMEM((PCH, 128), jnp.float32),
                       pltpu.VMEM((16, SS), jnp.float32),
                       pltpu.SemaphoreType.DMA],
    )(_sc_scatter_kernel)
    return run(pf16, key)


def kernel(voxels, coords, voxel_num_points, W, gamma, beta):
    pf, key = _pfn(voxels, coords, voxel_num_points, W, gamma, beta)
    canvas = _sc_scatter(pf, key)
    return canvas.reshape(2, CO, NY, NX)


# speedup vs baseline: 2.0446x; 2.0446x over previous
import functools

import jax
import jax.numpy as jnp
from jax import lax
from jax.experimental import pallas as pl
from jax.experimental.pallas import tpu as pltpu
from jax.experimental.pallas import tpu_sc as plsc

NX, NY = 432, 496
S = NX * NY
NV, MP, CP = 16000, 32, 4
CO = 64
EPS = 1e-3
VX, VY, VZ = 0.16, 0.16, 4.0
XOFF = VX / 2 + 0.0
YOFF = VY / 2 + (-39.68)
ZOFF = VZ / 2 + (-3.0)

BLK = 200
NBLK = NV // BLK

BANDS = 62
SS = S // BANDS
NTILE = 32
SCCH = 8000
PCH = 256
LISTCAP = 16128
YB = 8
HIGH = lax.Precision.HIGHEST


def _features(vox, coords, nump):
    if nump.ndim == 2:
        nump = nump[:, 0]
    numf = nump.astype(jnp.float32)
    xyz = vox[:, :, :3]
    pmean = jnp.sum(xyz, axis=1, keepdims=True) / numf[:, None, None]
    f_cluster = xyz - pmean
    cf = coords.astype(jnp.float32)
    cen = jnp.stack([cf[:, 3] * VX + XOFF,
                     cf[:, 2] * VY + YOFF,
                     cf[:, 1] * VZ + ZOFF], axis=-1)
    f_center = xyz - cen[:, None, :]
    f = jnp.concatenate([vox, f_cluster, f_center], axis=-1)
    msk = nump[:, None] > lax.broadcasted_iota(jnp.int32, (1, MP), 1)
    return f * msk.astype(jnp.float32)[:, :, None]


def _stats_kernel(vox_ref, coords_ref, nump_ref, g_ref):
    f = _features(vox_ref[...], coords_ref[...], nump_ref[...])
    ones = jnp.ones(f.shape[:2] + (1,), jnp.float32)
    f1 = jnp.concatenate([f, ones], axis=-1).reshape(BLK * MP, 11)
    g = lax.dot_general(f1, f1, (((0,), (0,)), ((), ())),
                        precision=HIGH, preferred_element_type=jnp.float32)

    @pl.when(pl.program_id(0) == 0)
    def _():
        g_ref[...] = jnp.zeros_like(g_ref)

    g_ref[...] += g


def _apply_kernel(vox_ref, coords_ref, nump_ref, g_ref, w_ref, gam_ref,
                  bet_ref, pf_ref, key_ref):
    G = g_ref[...]
    W = w_ref[...]
    cnt = G[10, 10]
    svec = G[10, :10]
    mu = jnp.sum(W * svec[None, :], axis=1) / cnt
    WG = lax.dot_general(W, G[:10, :10], (((1,), (0,)), ((), ())),
                         precision=HIGH, preferred_element_type=jnp.float32)
    e2 = jnp.sum(WG * W, axis=1) / cnt
    var = e2 - mu * mu
    scale = gam_ref[...] * lax.rsqrt(var + EPS)
    shift = bet_ref[...] - mu * scale

    f = _features(vox_ref[...], coords_ref[...], nump_ref[...])
    h = lax.dot_general(f.reshape(BLK * MP, 10), W, (((1,), (1,)), ((), ())),
                        precision=HIGH, preferred_element_type=jnp.float32)
    h = h.reshape(BLK, MP, CO)
    h = jnp.maximum(h * scale[None, None, :] + shift[None, None, :], 0.0)
    pf_ref[...] = jnp.concatenate(
        [jnp.max(h, axis=1), jnp.zeros((BLK, 128 - CO), jnp.float32)], axis=1)
    c = coords_ref[...]
    key_ref[...] = (c[:, 0] * S + c[:, 1] + c[:, 2] * NX + c[:, 3])[:, None]


def _pfn(voxels, coords, voxel_num_points, W, gamma, beta):
    G = pl.pallas_call(
        _stats_kernel,
        grid=(NBLK,),
        in_specs=[pl.BlockSpec((BLK, MP, CP), lambda i: (i, 0, 0)),
                  pl.BlockSpec((BLK, 4), lambda i: (i, 0)),
                  pl.BlockSpec((BLK, 1), lambda i: (i, 0))],
        out_specs=pl.BlockSpec((11, 11), lambda i: (0, 0)),
        out_shape=jax.ShapeDtypeStruct((11, 11), jnp.float32),
    )(voxels, coords, voxel_num_points.reshape(NV, 1))

    pf, key = pl.pallas_call(
        _apply_kernel,
        grid=(NBLK,),
        in_specs=[pl.BlockSpec((BLK, MP, CP), lambda i: (i, 0, 0)),
                  pl.BlockSpec((BLK, 4), lambda i: (i, 0)),
                  pl.BlockSpec((BLK, 1), lambda i: (i, 0)),
                  pl.BlockSpec((11, 11), lambda i: (0, 0)),
                  pl.BlockSpec((CO, 10), lambda i: (0, 0)),
                  pl.BlockSpec((CO,), lambda i: (0,)),
                  pl.BlockSpec((CO,), lambda i: (0,))],
        out_specs=[pl.BlockSpec((BLK, 128), lambda i: (i, 0)),
                   pl.BlockSpec((BLK, 1), lambda i: (i, 0))],
        out_shape=[jax.ShapeDtypeStruct((NV, 128), jnp.float32),
                   jax.ShapeDtypeStruct((NV, 1), jnp.int32)],
    )(voxels, coords, voxel_num_points.reshape(NV, 1), G, W, gamma, beta)
    return pf, key.reshape(NV)


def _sc_scatter_kernel(pf_hbm, key_hbm, out_hbm, key_v, list_v,
                       idx_v, stage_v, buf_v, sem):
    wid = lax.axis_index("s") * 2 + lax.axis_index("c")
    iota = lax.iota(jnp.int32, 16)
    z16 = jnp.zeros((16,), jnp.float32)

    for c in range(16):
        for y in range(YB):
            def zrow(j, _, c=c, y=y):
                buf_v[c, y, pl.ds(j * 16, 16)] = z16
                return 0
            lax.fori_loop(0, NX // 16, zrow, 0)

    def bin_body(t, _):
        bin_id = wid + t * NTILE

        @pl.when(bin_id < 2 * BANDS)
        def _():
            _do_bin(bin_id, pf_hbm, key_hbm, out_hbm, key_v, list_v,
                    idx_v, stage_v, buf_v, sem)
        return 0

    lax.fori_loop(0, pl.cdiv(2 * BANDS, NTILE), bin_body, 0)


def _do_bin(bin_id, pf_hbm, key_hbm, out_hbm, key_v, list_v,
            idx_v, stage_v, buf_v, sem):
        iota = lax.iota(jnp.int32, 16)
        z16 = jnp.zeros((16,), jnp.float32)
        b = bin_id // BANDS
        band = bin_id % BANDS
        base = b * S + band * SS

        def scan_chunk(kc, cnt):
            pltpu.sync_copy(key_hbm.at[pl.ds(kc * SCCH, SCCH)], key_v)

            def scan_vec(i, cnt):
                kv = key_v[pl.ds(i * 16, 16)]
                rel = kv - base
                m = (rel >= 0) & (rel < SS)
                idvec = kc * SCCH + i * 16 + iota
                mi = m.astype(jnp.int32)
                pos = cnt - 1 + plsc.cumsum(mi)
                plsc.store_scatter(list_v, [pos], idvec * 4096 + rel, mask=m)
                return cnt + jnp.sum(mi)

            return lax.fori_loop(0, SCCH // 16, scan_vec, cnt)

        n = lax.fori_loop(0, NV // SCCH, scan_chunk, 0)
        nch = (n + PCH - 1) // PCH

        def cg_body(cg, _):
            def chunk_body(ch, _):
                c0 = ch * PCH
                m = jnp.minimum(PCH, n - c0)

                @pl.when((cg == 0) | (nch > 1))
                def _():
                    def bidx(j, _):
                        v = list_v[pl.ds(c0 + j * 16, 16)] >> 12
                        v = jnp.minimum(jnp.maximum(v, 0), NV - 1)
                        idx_v[pl.ds(j * 16, 16)] = v
                        return 0

                    lax.fori_loop(0, PCH // 16, bidx, 0)
                    pltpu.async_copy(pf_hbm.at[idx_v], stage_v, sem).wait()

                def pillar(p, _):
                    osp = plsc.load_gather(
                        list_v, [jnp.full((16,), c0 + p, jnp.int32)]) & 4095
                    vy = osp // NX
                    vx = osp - vy * NX
                    row = plsc.load_gather(
                        stage_v, [jnp.full((16,), p, jnp.int32), cg * 16 + iota])
                    plsc.store_scatter(buf_v, [iota, vy, vx], row)
                    return 0

                lax.fori_loop(0, m, pillar, 0)
                return 0

            lax.fori_loop(0, nch, chunk_body, 0)

            pltpu.sync_copy(
                buf_v,
                out_hbm.at[b, pl.ds(cg * 16, 16), pl.ds(band * YB, YB), :])

            def rchunk(ch, _):
                c0 = ch * PCH
                m = jnp.minimum(PCH, n - c0)

                def rpillar(p, _):
                    osp = plsc.load_gather(
                        list_v, [jnp.full((16,), c0 + p, jnp.int32)]) & 4095
                    vy = osp // NX
                    vx = osp - vy * NX
                    plsc.store_scatter(buf_v, [iota, vy, vx], z16)
                    return 0

                lax.fori_loop(0, m, rpillar, 0)
                return 0

            lax.fori_loop(0, nch, rchunk, 0)
            return 0

        lax.fori_loop(0, 4, cg_body, 0)


def _sc_scatter(pf16, key):
    mesh = plsc.VectorSubcoreMesh(core_axis_name="c", subcore_axis_name="s")
    run = functools.partial(
        pl.kernel, mesh=mesh,
        compiler_params=pltpu.CompilerParams(needs_layout_passes=False),
        out_type=jax.ShapeDtypeStruct((2, CO, NY, NX), jnp.float32),
        scratch_types=[pltpu.VMEM((SCCH,), jnp.int32),
                       pltpu.VMEM((LISTCAP,), jnp.int32),
                       pltpu.VMEM((PCH,), jnp.int32),
                       pltpu.VMEM((PCH, 128), jnp.float32),
                       pltpu.VMEM((16, YB, NX), jnp.float32),
                       pltpu.SemaphoreType.DMA],
    )(_sc_scatter_kernel)
    return run(pf16, key)


def kernel(voxels, coords, voxel_num_points, W, gamma, beta):
    pf, key = _pfn(voxels, coords, voxel_num_points, W, gamma, beta)
    return _sc_scatter(pf, key)


# speedup vs baseline: 2.8039x; 1.3714x over previous
import functools

import jax
import jax.numpy as jnp
from jax import lax
from jax.experimental import pallas as pl
from jax.experimental.pallas import tpu as pltpu
from jax.experimental.pallas import tpu_sc as plsc

NX, NY = 432, 496
S = NX * NY
NV, MP, CP = 16000, 32, 4
CO = 64
EPS = 1e-3
VX, VY, VZ = 0.16, 0.16, 4.0
XOFF = VX / 2 + 0.0
YOFF = VY / 2 + (-39.68)
ZOFF = VZ / 2 + (-3.0)

BLK = 200
NBLK = NV // BLK

BANDS = 62
SS = S // BANDS
NTILE = 32
SCCH = 4000
PCH = 192
LISTCAP = 16128
YB = 8
TB = 4
HIGH = lax.Precision.HIGHEST


def _features(vox, coords, nump):
    if nump.ndim == 2:
        nump = nump[:, 0]
    numf = nump.astype(jnp.float32)
    xyz = vox[:, :, :3]
    pmean = jnp.sum(xyz, axis=1, keepdims=True) / numf[:, None, None]
    f_cluster = xyz - pmean
    cf = coords.astype(jnp.float32)
    cen = jnp.stack([cf[:, 3] * VX + XOFF,
                     cf[:, 2] * VY + YOFF,
                     cf[:, 1] * VZ + ZOFF], axis=-1)
    f_center = xyz - cen[:, None, :]
    f = jnp.concatenate([vox, f_cluster, f_center], axis=-1)
    msk = nump[:, None] > lax.broadcasted_iota(jnp.int32, (1, MP), 1)
    return f * msk.astype(jnp.float32)[:, :, None]


def _stats_kernel(vox_ref, coords_ref, nump_ref, g_ref):
    f = _features(vox_ref[...], coords_ref[...], nump_ref[...])
    ones = jnp.ones(f.shape[:2] + (1,), jnp.float32)
    f1 = jnp.concatenate([f, ones], axis=-1).reshape(BLK * MP, 11)
    g = lax.dot_general(f1, f1, (((0,), (0,)), ((), ())),
                        precision=HIGH, preferred_element_type=jnp.float32)

    @pl.when(pl.program_id(0) == 0)
    def _():
        g_ref[...] = jnp.zeros_like(g_ref)

    g_ref[...] += g


def _apply_kernel(vox_ref, coords_ref, nump_ref, g_ref, w_ref, gam_ref,
                  bet_ref, pf_ref, key_ref):
    G = g_ref[...]
    W = w_ref[...]
    cnt = G[10, 10]
    svec = G[10, :10]
    mu = jnp.sum(W * svec[None, :], axis=1) / cnt
    WG = lax.dot_general(W, G[:10, :10], (((1,), (0,)), ((), ())),
                         precision=HIGH, preferred_element_type=jnp.float32)
    e2 = jnp.sum(WG * W, axis=1) / cnt
    var = e2 - mu * mu
    scale = gam_ref[...] * lax.rsqrt(var + EPS)
    shift = bet_ref[...] - mu * scale

    f = _features(vox_ref[...], coords_ref[...], nump_ref[...])
    h = lax.dot_general(f.reshape(BLK * MP, 10), W, (((1,), (1,)), ((), ())),
                        precision=HIGH, preferred_element_type=jnp.float32)
    h = h.reshape(BLK, MP, CO)
    h = jnp.maximum(h * scale[None, None, :] + shift[None, None, :], 0.0)
    pf_ref[...] = jnp.concatenate(
        [jnp.max(h, axis=1), jnp.zeros((BLK, 128 - CO), jnp.float32)], axis=1)
    c = coords_ref[...]
    key_ref[...] = (c[:, 0] * S + c[:, 1] + c[:, 2] * NX + c[:, 3])[:, None]


def _pfn(voxels, coords, voxel_num_points, W, gamma, beta):
    G = pl.pallas_call(
        _stats_kernel,
        grid=(NBLK,),
        in_specs=[pl.BlockSpec((BLK, MP, CP), lambda i: (i, 0, 0)),
                  pl.BlockSpec((BLK, 4), lambda i: (i, 0)),
                  pl.BlockSpec((BLK, 1), lambda i: (i, 0))],
        out_specs=pl.BlockSpec((11, 11), lambda i: (0, 0)),
        out_shape=jax.ShapeDtypeStruct((11, 11), jnp.float32),
    )(voxels, coords, voxel_num_points.reshape(NV, 1))

    pf, key = pl.pallas_call(
        _apply_kernel,
        grid=(NBLK,),
        in_specs=[pl.BlockSpec((BLK, MP, CP), lambda i: (i, 0, 0)),
                  pl.BlockSpec((BLK, 4), lambda i: (i, 0)),
                  pl.BlockSpec((BLK, 1), lambda i: (i, 0)),
                  pl.BlockSpec((11, 11), lambda i: (0, 0)),
                  pl.BlockSpec((CO, 10), lambda i: (0, 0)),
                  pl.BlockSpec((CO,), lambda i: (0,)),
                  pl.BlockSpec((CO,), lambda i: (0,))],
        out_specs=[pl.BlockSpec((BLK, 128), lambda i: (i, 0)),
                   pl.BlockSpec((BLK, 1), lambda i: (i, 0))],
        out_shape=[jax.ShapeDtypeStruct((NV, 128), jnp.float32),
                   jax.ShapeDtypeStruct((NV, 1), jnp.int32)],
    )(voxels, coords, voxel_num_points.reshape(NV, 1), G, W, gamma, beta)
    return pf, key.reshape(NV)


def _sc_scatter_kernel(pf_hbm, key_hbm, out_hbm, key_v, list_v, bpos_v,
                       idx_v, stage_v, buf_v, sem):
    wid = lax.axis_index("s") * 2 + lax.axis_index("c")
    iota = lax.iota(jnp.int32, 16)
    z16 = jnp.zeros((16,), jnp.float32)
    tbase = wid * TB * SS

    for c in range(16):
        for y in range(YB):
            def zrow(j, _, c=c, y=y):
                buf_v[c, y, pl.ds(j * 16, 16)] = z16
                return 0
            lax.fori_loop(0, NX // 16, zrow, 0)

    def scan_chunk(kc, cnt):
        pltpu.sync_copy(key_hbm.at[pl.ds(kc * SCCH, SCCH)], key_v)

        def scan_vec(i, cnt):
            kv = key_v[pl.ds(i * 16, 16)]
            rel = kv - tbase
            m = (rel >= 0) & (rel < TB * SS)
            idvec = kc * SCCH + i * 16 + iota
            mi = m.astype(jnp.int32)
            pos = cnt - 1 + plsc.cumsum(mi)
            plsc.store_scatter(list_v, [pos], idvec * 16384 + rel, mask=m)
            return cnt + jnp.sum(mi)

        return lax.fori_loop(0, SCCH // 16, scan_vec, cnt)

    nt = lax.fori_loop(0, NV // SCCH, scan_chunk, 0)

    def bin_body(t, _):
        bin_id = wid * TB + t
        b = bin_id // BANDS
        band = bin_id % BANDS
        lo = t * SS

        def filt(i, cnt):
            pk = list_v[pl.ds(i * 16, 16)]
            rel = pk & 16383
            m = ((rel >= lo) & (rel < lo + SS)) & (i * 16 + iota < nt)
            mi = m.astype(jnp.int32)
            pos = cnt - 1 + plsc.cumsum(mi)
            plsc.store_scatter(bpos_v, [pos], i * 16 + iota, mask=m)
            return cnt + jnp.sum(mi)

        n = lax.fori_loop(0, pl.cdiv(nt, 16), filt, 0)
        nch = (n + PCH - 1) // PCH

        def cg_body(cg, _):
            def chunk_body(ch, _):
                c0 = ch * PCH
                m = jnp.minimum(PCH, n - c0)

                @pl.when((cg == 0) | (nch > 1))
                def _():
                    def bidx(j, _):
                        posv = bpos_v[pl.ds(c0 + j * 16, 16)]
                        posv = jnp.minimum(jnp.maximum(posv, 0), LISTCAP - 1)
                        pkv = plsc.load_gather(list_v, [posv]) >> 14
                        pkv = jnp.minimum(jnp.maximum(pkv, 0), NV - 1)
                        idx_v[pl.ds(j * 16, 16)] = pkv
                        return 0

                    lax.fori_loop(0, PCH // 16, bidx, 0)
                    pltpu.async_copy(pf_hbm.at[idx_v], stage_v, sem).wait()

                def pillar(p, _):
                    psp = plsc.load_gather(
                        bpos_v, [jnp.full((16,), c0 + p, jnp.int32)])
                    osp = (plsc.load_gather(list_v, [psp]) & 16383) - lo
                    vy = osp // NX
                    vx = osp - vy * NX
                    row = plsc.load_gather(
                        stage_v, [jnp.full((16,), p, jnp.int32), cg * 16 + iota])
                    plsc.store_scatter(buf_v, [iota, vy, vx], row)
                    return 0

                lax.fori_loop(0, m, pillar, 0)
                return 0

            lax.fori_loop(0, nch, chunk_body, 0)

            pltpu.sync_copy(
                buf_v,
                out_hbm.at[b, pl.ds(cg * 16, 16), pl.ds(band * YB, YB), :])
            return 0

        lax.fori_loop(0, 4, cg_body, 0)

        def rpillar(p, _):
            psp = plsc.load_gather(bpos_v, [jnp.full((16,), p, jnp.int32)])
            osp = (plsc.load_gather(list_v, [psp]) & 16383) - lo
            vy = osp // NX
            vx = osp - vy * NX
            plsc.store_scatter(buf_v, [iota, vy, vx], z16)
            return 0

        lax.fori_loop(0, n, rpillar, 0)
        return 0

    lax.fori_loop(0, TB, bin_body, 0)


def _sc_scatter(pf16, key):
    mesh = plsc.VectorSubcoreMesh(core_axis_name="c", subcore_axis_name="s")
    run = functools.partial(
        pl.kernel, mesh=mesh,
        compiler_params=pltpu.CompilerParams(needs_layout_passes=False),
        out_type=jax.ShapeDtypeStruct((2, CO, NY, NX), jnp.float32),
        scratch_types=[pltpu.VMEM((SCCH,), jnp.int32),
                       pltpu.VMEM((LISTCAP,), jnp.int32),
                       pltpu.VMEM((LISTCAP,), jnp.int32),
                       pltpu.VMEM((PCH,), jnp.int32),
                       pltpu.VMEM((PCH, 128), jnp.float32),
                       pltpu.VMEM((16, YB, NX), jnp.float32),
                       pltpu.SemaphoreType.DMA],
    )(_sc_scatter_kernel)
    return run(pf16, key)


def kernel(voxels, coords, voxel_num_points, W, gamma, beta):
    pf, key = _pfn(voxels, coords, voxel_num_points, W, gamma, beta)
    return _sc_scatter(pf, key)
